# Initial kernel scaffold; baseline (speedup 1.0000x reference)
import jax
import jax.numpy as jnp
from jax.experimental import pallas as pl


def kernel(student_emb, teacher_emb, centers):
    raise NotImplementedError("write your pallas kernel here")


# speedup vs baseline: 20.8868x; 20.8868x over previous
import functools

import jax
import jax.numpy as jnp
from jax import lax
from jax.experimental import pallas as pl
from jax.experimental.pallas import tpu as pltpu

_K = 8
_BETA = 0.5
_EPS = 1e-8


def _fused_body(s_blk_ref, t_blk_ref, c_blk_ref, s_full_ref, t_full_ref,
                c_full_ref, out_ref, inv_s_scr, inv_t_scr, nrb):
    b = pl.program_id(0)
    rb = pl.program_id(1)
    s_blk = s_blk_ref[0]
    t_blk = t_blk_ref[0]
    c_blk = c_blk_ref[0]
    R = s_blk.shape[0]
    P = s_full_ref.shape[1]
    D = s_blk.shape[1]
    ones_d = jnp.ones((1, D), jnp.float32)
    nt_dims = (((1,), (1,)), ((), ()))

    @pl.when(rb == 0)
    def _compute_full_norms():
        s_full = s_full_ref[0]
        t_full = t_full_ref[0]
        n2s = lax.dot_general(ones_d, s_full * s_full, nt_dims,
                              preferred_element_type=jnp.float32)
        n2t = lax.dot_general(ones_d, t_full * t_full, nt_dims,
                              preferred_element_type=jnp.float32)
        inv_s_scr[...] = 1.0 / jnp.maximum(jnp.sqrt(n2s), _EPS)
        inv_t_scr[...] = 1.0 / jnp.maximum(jnp.sqrt(n2t), _EPS)

    s_full = s_full_ref[0]
    t_full = t_full_ref[0]
    c_full = c_full_ref[0]

    inv_s_blk = 1.0 / jnp.maximum(
        jnp.sqrt(jnp.sum(s_blk * s_blk, axis=1, keepdims=True)), _EPS)
    inv_t_blk = 1.0 / jnp.maximum(
        jnp.sqrt(jnp.sum(t_blk * t_blk, axis=1, keepdims=True)), _EPS)

    gs = lax.dot_general(s_blk, s_full, nt_dims,
                         preferred_element_type=jnp.float32)
    gs = gs * inv_s_blk * inv_s_scr[...]
    gt = lax.dot_general(t_blk, t_full, nt_dims,
                         preferred_element_type=jnp.float32)
    gt = gt * inv_t_blk * inv_t_scr[...]

    ones_3 = jnp.ones((1, 3), jnp.float32)
    c_dims = (((1,), (1,)), ((), ()))
    dotc = lax.dot_general(c_blk, c_full, c_dims,
                           preferred_element_type=jnp.float32)
    n2c_blk = jnp.sum(c_blk * c_blk, axis=1, keepdims=True)
    n2c_full = lax.dot_general(ones_3, c_full * c_full, c_dims,
                               preferred_element_type=jnp.float32)
    d2 = n2c_blk + n2c_full - 2.0 * dotc

    col = lax.broadcasted_iota(jnp.int32, (R, P), 1)
    row = lax.broadcasted_iota(jnp.int32, (R, P), 0) + rb * R
    inf = jnp.float32(jnp.inf)
    d2 = jnp.where(col == row, inf, d2)

    big = jnp.int32(1 << 24)
    loss_acc = jnp.zeros((R, 1), jnp.float32)
    for _ in range(_K):
        m = jnp.min(d2, axis=1, keepdims=True)
        ismin = d2 <= m
        colsel = jnp.min(jnp.where(ismin, col, big), axis=1, keepdims=True)
        onehot = col == colsel
        cs = jnp.sum(jnp.where(onehot, gs, 0.0), axis=1, keepdims=True)
        ct = jnp.sum(jnp.where(onehot, gt, 0.0), axis=1, keepdims=True)
        d2 = jnp.where(onehot, inf, d2)
        ax = jnp.abs(cs - ct)
        loss_acc = loss_acc + jnp.where(
            ax < _BETA, 0.5 * ax * ax / _BETA, ax - 0.5 * _BETA)

    part = jnp.sum(loss_acc, axis=(0, 1), keepdims=True)

    @pl.when((b == 0) & (rb == 0))
    def _init():
        out_ref[...] = jnp.zeros((1, 1), jnp.float32)

    out_ref[...] += part


def kernel(student_emb, teacher_emb, centers):
    B, P, D = student_emb.shape
    R = min(256, P)
    nrb = P // R
    grid = (B, nrb)

    blk = lambda r: pl.BlockSpec((1, r, D), lambda b, rb: (b, rb if r != P else 0, 0))
    cblk = lambda r: pl.BlockSpec((1, r, 3), lambda b, rb: (b, rb if r != P else 0, 0))
    full = pl.BlockSpec((1, P, D), lambda b, rb: (b, 0, 0))
    cfull = pl.BlockSpec((1, P, 3), lambda b, rb: (b, 0, 0))

    out = pl.pallas_call(
        functools.partial(_fused_body, nrb=nrb),
        grid=grid,
        in_specs=[blk(R), blk(R), cblk(R), full, full, cfull],
        out_specs=pl.BlockSpec((1, 1), lambda b, rb: (0, 0)),
        out_shape=jax.ShapeDtypeStruct((1, 1), jnp.float32),
        scratch_shapes=[pltpu.VMEM((1, P), jnp.float32),
                        pltpu.VMEM((1, P), jnp.float32)],
        compiler_params=pltpu.CompilerParams(
            dimension_semantics=("arbitrary", "arbitrary")),
    )(student_emb, teacher_emb, centers, student_emb, teacher_emb, centers)

    K = min(_K + 1, P) - 1
    return out[0, 0] / jnp.float32(B * P * K)


# speedup vs baseline: 26.2308x; 1.2559x over previous
import jax
import jax.numpy as jnp
from jax import lax
from jax.experimental import pallas as pl
from jax.experimental.pallas import tpu as pltpu

_K = 8
_BETA = 0.5
_EPS = 1e-8


def _fused_body(c_blk_ref, s_full_ref, t_full_ref, c_full_ref, out_ref,
                s_scr, t_scr):
    b = pl.program_id(0)
    rb = pl.program_id(1)
    R = c_blk_ref.shape[1]
    P = s_full_ref.shape[1]
    D = s_full_ref.shape[2]
    ones_d = jnp.ones((1, D), jnp.float32)
    nt_dims = (((1,), (1,)), ((), ()))
    hi_mask = jnp.uint32(0xFFFF0000)

    @pl.when(rb == 0)
    def _normalize_full():
        for full_ref, scr in ((s_full_ref, s_scr), (t_full_ref, t_scr)):
            x = full_ref[0]
            n2 = lax.dot_general(x * x, ones_d, nt_dims,
                                 preferred_element_type=jnp.float32)
            inv = 1.0 / jnp.maximum(jnp.sqrt(n2), _EPS)
            scr[...] = (x * inv).astype(jnp.bfloat16)

    s_blk = s_scr[pl.ds(rb * R, R), :]
    t_blk = t_scr[pl.ds(rb * R, R), :]
    gs = lax.dot_general(s_blk, s_scr[...], nt_dims,
                         preferred_element_type=jnp.float32)
    gt = lax.dot_general(t_blk, t_scr[...], nt_dims,
                         preferred_element_type=jnp.float32)

    rnd = jnp.uint32(0x8000)
    pk_hi = (lax.bitcast_convert_type(gs, jnp.uint32) + rnd) & hi_mask
    pk_lo = (lax.bitcast_convert_type(gt, jnp.uint32) + rnd) >> 16
    pk = lax.bitcast_convert_type(pk_hi | pk_lo, jnp.int32)

    c_blk = c_blk_ref[0]
    c_full = c_full_ref[0]
    ones_3 = jnp.ones((1, 3), jnp.float32)
    dotc = lax.dot_general(c_blk, c_full, nt_dims,
                           preferred_element_type=jnp.float32)
    n2c_blk = jnp.sum(c_blk * c_blk, axis=1, keepdims=True)
    n2c_full = lax.dot_general(ones_3, c_full * c_full, nt_dims,
                               preferred_element_type=jnp.float32)
    d2 = jnp.maximum(n2c_blk + n2c_full - 2.0 * dotc, 0.0)

    col = lax.broadcasted_iota(jnp.int32, (R, P), 1)
    row = lax.broadcasted_iota(jnp.int32, (R, P), 0) + rb * R
    d2 = jnp.where(col == row, jnp.float32(jnp.inf), d2)

    ck = lax.bitcast_convert_type(
        (lax.bitcast_convert_type(d2, jnp.uint32) & jnp.uint32(0xFFFFFC00))
        | lax.bitcast_convert_type(col, jnp.uint32).astype(jnp.uint32),
        jnp.int32)

    imax = jnp.int32(2**31 - 1)
    loss_acc = jnp.zeros((R, 1), jnp.float32)
    for _ in range(_K):
        m = jnp.min(ck, axis=1, keepdims=True)
        onehot = ck == m
        sel = jnp.sum(jnp.where(onehot, pk, 0), axis=1, keepdims=True)
        ck = jnp.where(onehot, imax, ck)
        selu = lax.bitcast_convert_type(sel, jnp.uint32)
        cs = lax.bitcast_convert_type(selu & hi_mask, jnp.float32)
        ct = lax.bitcast_convert_type(selu << 16, jnp.float32)
        ax = jnp.abs(cs - ct)
        loss_acc = loss_acc + jnp.where(
            ax < _BETA, 0.5 * ax * ax / _BETA, ax - 0.5 * _BETA)

    part = jnp.sum(loss_acc, axis=(0, 1), keepdims=True)

    @pl.when((b == 0) & (rb == 0))
    def _init():
        out_ref[...] = jnp.zeros((1, 1), jnp.float32)

    out_ref[...] += part


def kernel(student_emb, teacher_emb, centers):
    B, P, D = student_emb.shape
    R = min(256, P)
    nrb = P // R
    grid = (B, nrb)

    cblk = pl.BlockSpec((1, R, 3), lambda b, rb: (b, rb if R != P else 0, 0))
    full = pl.BlockSpec((1, P, D), lambda b, rb: (b, 0, 0))
    cfull = pl.BlockSpec((1, P, 3), lambda b, rb: (b, 0, 0))

    out = pl.pallas_call(
        _fused_body,
        grid=grid,
        in_specs=[cblk, full, full, cfull],
        out_specs=pl.BlockSpec((1, 1), lambda b, rb: (0, 0)),
        out_shape=jax.ShapeDtypeStruct((1, 1), jnp.float32),
        scratch_shapes=[pltpu.VMEM((P, D), jnp.bfloat16),
                        pltpu.VMEM((P, D), jnp.bfloat16)],
        compiler_params=pltpu.CompilerParams(
            dimension_semantics=("arbitrary", "arbitrary")),
    )(centers, student_emb, teacher_emb, centers)

    K = min(_K + 1, P) - 1
    return out[0, 0] / jnp.float32(B * P * K)


# speedup vs baseline: 33.9895x; 1.2958x over previous
import jax
import jax.numpy as jnp
from jax import lax
from jax.experimental import pallas as pl
from jax.experimental.pallas import tpu as pltpu

_K = 8
_BETA = 0.5
_EPS = 1e-8


def _fused_body(c_blk_ref, s_full_ref, t_full_ref, c_full_ref, out_ref,
                s_scr, t_scr):
    b = pl.program_id(0)
    rb = pl.program_id(1)
    R = c_blk_ref.shape[1]
    P = s_full_ref.shape[1]
    D = s_full_ref.shape[2]
    ones_d = jnp.ones((1, D), jnp.float32)
    nt_dims = (((1,), (1,)), ((), ()))
    hi_mask = jnp.uint32(0xFFFF0000)

    @pl.when(rb == 0)
    def _normalize_full():
        for full_ref, scr in ((s_full_ref, s_scr), (t_full_ref, t_scr)):
            x = full_ref[0]
            n2 = lax.dot_general(x * x, ones_d, nt_dims,
                                 preferred_element_type=jnp.float32)
            inv = 1.0 / jnp.maximum(jnp.sqrt(n2), _EPS)
            scr[...] = (x * inv).astype(jnp.bfloat16)

    s_blk = s_scr[pl.ds(rb * R, R), :]
    t_blk = t_scr[pl.ds(rb * R, R), :]
    gs = lax.dot_general(s_blk, s_scr[...], nt_dims,
                         preferred_element_type=jnp.float32)
    gt = lax.dot_general(t_blk, t_scr[...], nt_dims,
                         preferred_element_type=jnp.float32)

    c_blk = c_blk_ref[0]
    c_full = c_full_ref[0]
    ones_3 = jnp.ones((1, 3), jnp.float32)
    dotc = lax.dot_general(c_blk, c_full, nt_dims,
                           preferred_element_type=jnp.float32)
    n2c_blk = jnp.sum(c_blk * c_blk, axis=1, keepdims=True)
    n2c_full = lax.dot_general(ones_3, c_full * c_full, nt_dims,
                               preferred_element_type=jnp.float32)
    d2 = jnp.maximum(n2c_blk + n2c_full - 2.0 * dotc, 0.0)

    col = lax.broadcasted_iota(jnp.int32, (R, P), 1)
    row = lax.broadcasted_iota(jnp.int32, (R, P), 0) + rb * R
    d2 = jnp.where(col == row, jnp.float32(jnp.inf), d2)

    ck = lax.bitcast_convert_type(
        (lax.bitcast_convert_type(d2, jnp.uint32) & jnp.uint32(0xFFFFFC00))
        | lax.bitcast_convert_type(col, jnp.uint32).astype(jnp.uint32),
        jnp.int32)

    imax = jnp.int32(2**31 - 1)
    for _ in range(_K):
        m = jnp.min(ck, axis=1, keepdims=True)
        ck = jnp.where(ck == m, imax, ck)
    mask8 = ck == imax

    ax = jnp.abs(gs - gt)
    f = jnp.where(ax < _BETA, 0.5 * ax * ax / _BETA, ax - 0.5 * _BETA)
    part = jnp.sum(jnp.where(mask8, f, 0.0), axis=(0, 1),
                   keepdims=True)

    @pl.when((b == 0) & (rb == 0))
    def _init():
        out_ref[...] = jnp.zeros((1, 1), jnp.float32)

    out_ref[...] += part


def kernel(student_emb, teacher_emb, centers):
    B, P, D = student_emb.shape
    R = min(256, P)
    nrb = P // R
    grid = (B, nrb)

    cblk = pl.BlockSpec((1, R, 3), lambda b, rb: (b, rb if R != P else 0, 0))
    full = pl.BlockSpec((1, P, D), lambda b, rb: (b, 0, 0))
    cfull = pl.BlockSpec((1, P, 3), lambda b, rb: (b, 0, 0))

    out = pl.pallas_call(
        _fused_body,
        grid=grid,
        in_specs=[cblk, full, full, cfull],
        out_specs=pl.BlockSpec((1, 1), lambda b, rb: (0, 0)),
        out_shape=jax.ShapeDtypeStruct((1, 1), jnp.float32),
        scratch_shapes=[pltpu.VMEM((P, D), jnp.bfloat16),
                        pltpu.VMEM((P, D), jnp.bfloat16)],
        compiler_params=pltpu.CompilerParams(
            dimension_semantics=("arbitrary", "arbitrary")),
    )(centers, student_emb, teacher_emb, centers)

    K = min(_K + 1, P) - 1
    return out[0, 0] / jnp.float32(B * P * K)
